# baseline (speedup 1.0000x reference)
import jax
import jax.numpy as jnp
from jax.experimental import pallas as pl
from jax.experimental.pallas import tpu as pltpu

_BN = 512
_HK = 1024


def _mm_kernel(x_hbm, w_ref, o_ref, land_ref, xb_ref, sem0, sem1):
    j = pl.program_id(0)

    copy0 = pltpu.make_async_copy(
        x_hbm.at[:, pl.ds(0, _HK)], land_ref.at[0], sem0)
    copy1 = pltpu.make_async_copy(
        x_hbm.at[:, pl.ds(_HK, _HK)], land_ref.at[1], sem1)

    @pl.when(j == 0)
    def _():
        copy0.start()
        copy1.start()
        copy0.wait()
        xb_ref[:, :_HK] = land_ref[0].astype(jnp.bfloat16)

    wb = w_ref[...].astype(jnp.bfloat16)
    dims = (((1,), (1,)), ((), ()))
    dA = jax.lax.dot_general(
        xb_ref[:, :_HK], wb[:, :_HK], dims,
        preferred_element_type=jnp.float32)

    @pl.when(j == 0)
    def _():
        copy1.wait()
        xb_ref[:, _HK:] = land_ref[1].astype(jnp.bfloat16)

    dB = jax.lax.dot_general(
        xb_ref[:, _HK:], wb[:, _HK:], dims,
        preferred_element_type=jnp.float32)
    o_ref[...] = dA + dB


def kernel(x, weight):
    M, K = x.shape
    N, _ = weight.shape
    return pl.pallas_call(
        _mm_kernel,
        grid=(N // _BN,),
        in_specs=[
            pl.BlockSpec(memory_space=pl.ANY),
            pl.BlockSpec((_BN, K), lambda j: (j, 0)),
        ],
        out_specs=pl.BlockSpec((M, _BN), lambda j: (0, j)),
        out_shape=jax.ShapeDtypeStruct((M, N), jnp.float32),
        scratch_shapes=[
            pltpu.VMEM((2, M, _HK), jnp.float32),
            pltpu.VMEM((M, K), jnp.bfloat16),
            pltpu.SemaphoreType.DMA,
            pltpu.SemaphoreType.DMA,
        ],
    )(x, weight)


# speedup vs baseline: 1.1143x; 1.1143x over previous
import jax
import jax.numpy as jnp
from jax.experimental import pallas as pl

_BN = 512
_NC = 4


def _mm_kernel(x0_ref, x1_ref, x2_ref, x3_ref, w_ref, o_ref):
    wb = w_ref[...].astype(jnp.bfloat16)
    cm = x0_ref.shape[0]
    for c, xc in enumerate((x0_ref, x1_ref, x2_ref, x3_ref)):
        o_ref[c * cm:(c + 1) * cm, :] = jax.lax.dot_general(
            xc[...].astype(jnp.bfloat16), wb, (((1,), (1,)), ((), ())),
            preferred_element_type=jnp.float32)


def kernel(x, weight):
    M, K = x.shape
    N, _ = weight.shape
    cm = M // _NC
    x_specs = [
        pl.BlockSpec((cm, K), (lambda j, c=c: (c, 0))) for c in range(_NC)
    ]
    return pl.pallas_call(
        _mm_kernel,
        grid=(N // _BN,),
        in_specs=x_specs + [pl.BlockSpec((_BN, K), lambda j: (j, 0))],
        out_specs=pl.BlockSpec((M, _BN), lambda j: (0, j)),
        out_shape=jax.ShapeDtypeStruct((M, N), jnp.float32),
    )(x, x, x, x, weight)
